# baseline (speedup 1.0000x reference)
import functools

import jax
import jax.numpy as jnp
from jax import lax
from jax.experimental import pallas as pl
from jax.experimental.pallas import tpu as pltpu
from jax.experimental.pallas import tpu_sc as plsc

B, Z, H, W, C = 1, 4, 32, 64, 256
WZ, WH, WW = 2, 8, 8
NH = 8
E = 64
ATTR = 96
AH = 128
HID = 1024
N = B * Z * H * W
CAP = int(1.25 * N / E)
HD = C // NH
WSZ = WZ * WH * WW
NSLOT = E * CAP

NWORK = 32
CHUNK = N // NWORK
BUF_ROWS = NSLOT + NWORK
RCH = 512
NRCH = N // RCH
TB = 1024


NWPP = H // WH
RB = WZ * H * WW


def _attn_body(x_ref, af_ref, g_ref, b_ref, wqkv_ref, bqkv_ref,
               wa1_ref, ba1_ref, wa2_ref, ba2_ref, wr1c_ref, wr1a_ref,
               br1_ref, tril_ref, o_ref, a2_ref, dst_ref, scale_ref,
               zl_ref, bl_ref, lgacc_ref):
    ones_c8 = jnp.full((C, 8), 1.0 / C, jnp.float32)
    ones_8c = jnp.full((8, C), 1.0 / 8.0, jnp.float32)
    ib = lax.broadcasted_iota(jnp.int32, (NH * WSZ, NH), 0) // WSZ
    jb = lax.broadcasted_iota(jnp.int32, (NH * WSZ, NH), 1)
    onesbd = (ib == jb).astype(jnp.float32)
    ih = lax.broadcasted_iota(jnp.int32, (NH, C), 0)
    jh = lax.broadcasted_iota(jnp.int32, (NH, C), 1) // HD
    rhot = (ih == jh).astype(jnp.float32)

    xb = x_ref[...].reshape(RB, C)
    m8 = jnp.dot(xb, ones_c8, preferred_element_type=jnp.float32)
    e8 = jnp.dot(xb * xb, ones_c8, preferred_element_type=jnp.float32)
    r8 = lax.rsqrt(e8 - m8 * m8 + 1e-5)
    mr = jnp.dot(m8, ones_8c, preferred_element_type=jnp.float32)
    rr = jnp.dot(r8, ones_8c, preferred_element_type=jnp.float32)
    xn = (xb - mr) * rr * g_ref[...] + b_ref[...]
    qkv = jnp.dot(xn, wqkv_ref[...], preferred_element_type=jnp.float32)
    qkv = qkv + bqkv_ref[...]
    q = qkv[:, :C] * (HD ** -0.5)
    k = qkv[:, C:2 * C]
    vv = qkv[:, 2 * C:]
    chunks = [None] * (2 * NWPP)
    for w in range(NWPP):
        r0 = slice(64 * w, 64 * w + 64)
        r1 = slice(RB // 2 + 64 * w, RB // 2 + 64 * w + 64)
        qw = jnp.concatenate([q[r0], q[r1]], axis=0)
        kw = jnp.concatenate([k[r0], k[r1]], axis=0)
        vw = jnp.concatenate([vv[r0], vv[r1]], axis=0)
        exps = []
        outs = []
        for h in range(NH):
            sl = slice(h * HD, (h + 1) * HD)
            qh, kh, vh = qw[:, sl], kw[:, sl], vw[:, sl]
            s = lax.dot_general(qh, kh, (((1,), (1,)), ((), ())),
                                preferred_element_type=jnp.float32)
            eh = jnp.exp(s)
            exps.append(eh)
            outs.append(jnp.dot(eh, vh, preferred_element_type=jnp.float32))
        ecat = jnp.concatenate(exps, axis=1)
        denom = jnp.dot(ecat, onesbd, preferred_element_type=jnp.float32)
        rep = jnp.dot(1.0 / denom, rhot, preferred_element_type=jnp.float32)
        ow = jnp.concatenate(outs, axis=1) * rep
        chunks[w] = ow[:64]
        chunks[NWPP + w] = ow[64:]
    o = jnp.concatenate(chunks, axis=0)
    o_ref[...] = o.reshape(x_ref.shape)
    af = af_ref[...].reshape(RB, ATTR)
    a1 = jax.nn.gelu(jnp.dot(af, wa1_ref[...], preferred_element_type=jnp.float32)
                     + ba1_ref[...])
    a2 = jax.nn.gelu(jnp.dot(af, wa2_ref[...], preferred_element_type=jnp.float32)
                     + ba2_ref[...])
    a2_ref[...] = a2.reshape(a2_ref.shape)
    lg1 = (jnp.dot(o, wr1c_ref[...], preferred_element_type=jnp.float32)
           + jnp.dot(a1, wr1a_ref[...], preferred_element_type=jnp.float32)
           + br1_ref[...])
    i = pl.program_id(0)
    j = pl.program_id(1)
    lgacc_ref[pl.ds(WZ * i, WZ), :, pl.ds(WW * j, WW), :] = (
        lg1.reshape(WZ, H, WW, E))

    @pl.when(jnp.logical_and(i == Z // WZ - 1, j == W // WW - 1))
    def _():
        lg = lgacc_ref[...].reshape(N, E)
        _route_math(lg, tril_ref[...], dst_ref, scale_ref, zl_ref, bl_ref)


def _window_attention(x, attr, n1_g, n1_b, w_qkv, b_qkv,
                      wa1, ba1, wa2, ba2, wr1, br1):
    wr1c, wr1a = wr1[:C], wr1[C:]
    grid = (Z // WZ, W // WW)
    blk = (1, WZ, H, WW, C)
    return pl.pallas_call(
        _attn_body,
        grid=grid,
        in_specs=[
            pl.BlockSpec(blk, lambda i, j: (0, i, 0, j, 0)),
            pl.BlockSpec((1, WZ, H, WW, ATTR), lambda i, j: (0, i, 0, j, 0)),
            pl.BlockSpec((1, C), lambda i, j: (0, 0)),
            pl.BlockSpec((1, C), lambda i, j: (0, 0)),
            pl.BlockSpec((C, 3 * C), lambda i, j: (0, 0)),
            pl.BlockSpec((1, 3 * C), lambda i, j: (0, 0)),
            pl.BlockSpec((ATTR, AH), lambda i, j: (0, 0)),
            pl.BlockSpec((1, AH), lambda i, j: (0, 0)),
            pl.BlockSpec((ATTR, AH), lambda i, j: (0, 0)),
            pl.BlockSpec((1, AH), lambda i, j: (0, 0)),
            pl.BlockSpec((C, E), lambda i, j: (0, 0)),
            pl.BlockSpec((AH, E), lambda i, j: (0, 0)),
            pl.BlockSpec((1, E), lambda i, j: (0, 0)),
            pl.BlockSpec((RCH, RCH), lambda i, j: (0, 0)),
        ],
        out_specs=[
            pl.BlockSpec(blk, lambda i, j: (0, i, 0, j, 0)),
            pl.BlockSpec((1, WZ, H, WW, AH), lambda i, j: (0, i, 0, j, 0)),
            pl.BlockSpec((N, 1), lambda i, j: (0, 0)),
            pl.BlockSpec((N, 1), lambda i, j: (0, 0)),
            pl.BlockSpec((1, 1), lambda i, j: (0, 0)),
            pl.BlockSpec((1, 1), lambda i, j: (0, 0)),
        ],
        out_shape=[
            jax.ShapeDtypeStruct((B, Z, H, W, C), jnp.float32),
            jax.ShapeDtypeStruct((B, Z, H, W, AH), jnp.float32),
            jax.ShapeDtypeStruct((N, 1), jnp.int32),
            jax.ShapeDtypeStruct((N, 1), jnp.float32),
            jax.ShapeDtypeStruct((1, 1), jnp.float32),
            jax.ShapeDtypeStruct((1, 1), jnp.float32),
        ],
        scratch_shapes=[pltpu.VMEM((Z, H, W, E), jnp.float32)],
    )(x, attr, n1_g.reshape(1, C), n1_b.reshape(1, C), w_qkv,
      b_qkv.reshape(1, 3 * C), wa1, ba1.reshape(1, AH), wa2,
      ba2.reshape(1, AH), wr1c, wr1a, br1.reshape(1, E), _tril())


def _route_math(lg, tril, dst_ref, scale_ref, zl_ref, bl_ref):
    m = jnp.max(lg, -1, keepdims=True)
    ex = jnp.exp(lg - m)
    se = jnp.sum(ex, -1, keepdims=True)
    probs = ex / se
    lse = jnp.log(se) + m
    zl_ref[...] = jnp.sum(lse * lse, 0, keepdims=True) * (1.0 / N)
    p = jnp.max(probs, -1, keepdims=True)
    lanes = lax.broadcasted_iota(jnp.int32, (N, E), 1)
    eidx = jnp.min(jnp.where(probs >= p, lanes, E), -1, keepdims=True)
    oh = (lanes == eidx).astype(jnp.float32)
    bl_ref[...] = (float(E) / (float(N) * float(N))) * jnp.sum(
        jnp.sum(oh, 0, keepdims=True) * jnp.sum(probs, 0, keepdims=True),
        1, keepdims=True)
    scale_ref[...] = p
    ones_e8 = jnp.ones((E, 8), jnp.float32)
    off = jnp.zeros((1, E), jnp.float32)
    for cidx in range(NRCH):
        rows = slice(cidx * RCH, (cidx + 1) * RCH)
        ohc = oh[rows]
        cum = jnp.dot(tril, ohc, preferred_element_type=jnp.float32) + off
        pos8 = jnp.dot(cum * ohc, ones_e8,
                       preferred_element_type=jnp.float32)
        pos = pos8[:, :1] - 1.0
        keep = pos < float(CAP)
        posi = pos.astype(jnp.int32)
        ec = eidx[rows]
        half = (lax.broadcasted_iota(jnp.int32, (RCH, 1), 0) >= CHUNK
                ).astype(jnp.int32)
        dst = jnp.where(keep, ec * CAP + posi, NSLOT + 2 * cidx + half)
        dst_ref[rows, :] = dst
        off = cum[RCH - 1:RCH, :]


_TRIL = None


def _tril():
    global _TRIL
    if _TRIL is None:
        _TRIL = jnp.tril(jnp.ones((RCH, RCH), jnp.float32))
    return _TRIL


_SC_MESH = None


def _sc_mesh():
    global _SC_MESH
    if _SC_MESH is None:
        _SC_MESH = plsc.VectorSubcoreMesh(core_axis_name="c", subcore_axis_name="s")
    return _SC_MESH


@functools.partial(jax.jit)
def _sc_scatter(tok, dst2d):
    @functools.partial(
        pl.kernel,
        out_type=jax.ShapeDtypeStruct((BUF_ROWS, C), jnp.float32),
        mesh=_sc_mesh(),
        scratch_types=[
            pltpu.VMEM((2, 128), jnp.int32),
            pltpu.VMEM((128, C), jnp.float32),
            pltpu.SemaphoreType.DMA,
        ],
    )
    def scatter_k(tok_hbm, dst_hbm, buf_hbm, idx_v, rows_v, sem):
        wid = lax.axis_index("s") * 2 + lax.axis_index("c")
        base = wid * CHUNK
        pltpu.sync_copy(dst_hbm.at[pl.ds(wid * 2, 2)], idx_v)
        for j in range(2):
            pltpu.sync_copy(tok_hbm.at[pl.ds(base + j * 128, 128)], rows_v)
            pltpu.async_copy(rows_v, buf_hbm.at[idx_v.at[j]], sem).wait()

    return scatter_k(tok, dst2d)


@functools.partial(jax.jit)
def _sc_gather(buf, dst2d):
    @functools.partial(
        pl.kernel,
        out_type=jax.ShapeDtypeStruct((N, C), jnp.float32),
        mesh=_sc_mesh(),
        scratch_types=[
            pltpu.VMEM((2, 128), jnp.int32),
            pltpu.VMEM((128, C), jnp.float32),
            pltpu.SemaphoreType.DMA,
        ],
    )
    def gather_k(buf_hbm, dst_hbm, out_hbm, idx_v, rows_v, sem):
        wid = lax.axis_index("s") * 2 + lax.axis_index("c")
        base = wid * CHUNK
        pltpu.sync_copy(dst_hbm.at[pl.ds(wid * 2, 2)], idx_v)
        for j in range(2):
            pltpu.async_copy(buf_hbm.at[idx_v.at[j]], rows_v, sem).wait()
            pltpu.sync_copy(rows_v, out_hbm.at[pl.ds(base + j * 128, 128)])

    return gather_k(buf, dst2d)


def _exp1_body(buf_ref, wp_ref, bp_ref, ob_ref):
    e = pl.program_id(0)
    ob_ref[...] = (jnp.dot(buf_ref[...], wp_ref[0],
                           preferred_element_type=jnp.float32)
                   + bp_ref[pl.ds(e, 1), :])


def _expert_proj(buf, wp, bp):
    return pl.pallas_call(
        _exp1_body,
        grid=(E,),
        in_specs=[
            pl.BlockSpec((CAP, C), lambda e: (e, 0)),
            pl.BlockSpec((1, C, C), lambda e: (e, 0, 0)),
            pl.BlockSpec((E, C), lambda e: (0, 0)),
        ],
        out_specs=pl.BlockSpec((CAP, C), lambda e: (e, 0)),
        out_shape=jax.ShapeDtypeStruct((BUF_ROWS, C), jnp.float32),
    )(buf, wp, bp)


def _exp2_body(buf_ref, w1_ref, b1_ref, w2_ref, b2_ref, ob_ref):
    e = pl.program_id(0)
    h = jax.nn.gelu(jnp.dot(buf_ref[...], w1_ref[0],
                            preferred_element_type=jnp.float32)
                    + b1_ref[pl.ds(e, 1), :])
    ob_ref[...] = (jnp.dot(h, w2_ref[0], preferred_element_type=jnp.float32)
                   + b2_ref[pl.ds(e, 1), :])


def _expert_mlp(buf, w1, b1, w2, b2):
    return pl.pallas_call(
        _exp2_body,
        grid=(E,),
        in_specs=[
            pl.BlockSpec((CAP, C), lambda e: (e, 0)),
            pl.BlockSpec((1, C, HID), lambda e: (e, 0, 0)),
            pl.BlockSpec((E, HID), lambda e: (0, 0)),
            pl.BlockSpec((1, HID, C), lambda e: (e, 0, 0)),
            pl.BlockSpec((E, C), lambda e: (0, 0)),
        ],
        out_specs=pl.BlockSpec((CAP, C), lambda e: (e, 0)),
        out_shape=jax.ShapeDtypeStruct((BUF_ROWS, C), jnp.float32),
    )(buf, w1, b1, w2, b2)


def _mid_body(x_ref, g1_ref, dst_ref, scale_ref, a2_ref, n2g_ref, n2b_ref,
              wr2c_ref, wr2a_ref, br2_ref, tril_ref, x1_ref, xm_ref,
              dst2_ref, scale2_ref, zl_ref, bl_ref, lgacc_ref):
    kept = dst_ref[...] < NSLOT
    y1 = jnp.where(kept, g1_ref[...] * scale_ref[...], 0.0)
    x1 = x_ref[...] + y1
    x1_ref[...] = x1
    ones_c8 = jnp.full((C, 8), 1.0 / C, jnp.float32)
    ones_8c = jnp.full((8, C), 1.0 / 8.0, jnp.float32)
    m8 = jnp.dot(x1, ones_c8, preferred_element_type=jnp.float32)
    e8 = jnp.dot(x1 * x1, ones_c8, preferred_element_type=jnp.float32)
    r8 = lax.rsqrt(e8 - m8 * m8 + 1e-5)
    mr = jnp.dot(m8, ones_8c, preferred_element_type=jnp.float32)
    rr = jnp.dot(r8, ones_8c, preferred_element_type=jnp.float32)
    xm = (x1 - mr) * rr * n2g_ref[...] + n2b_ref[...]
    xm_ref[...] = xm
    lg2 = (jnp.dot(xm, wr2c_ref[...], preferred_element_type=jnp.float32)
           + jnp.dot(a2_ref[...], wr2a_ref[...], preferred_element_type=jnp.float32)
           + br2_ref[...])
    i = pl.program_id(0)
    lgacc_ref[pl.ds(i * TB, TB), :] = lg2

    @pl.when(i == N // TB - 1)
    def _():
        _route_math(lgacc_ref[...], tril_ref[...], dst2_ref, scale2_ref,
                    zl_ref, bl_ref)


def _mid(xf, g1, dst1, scale1, a2, n2_g, n2_b, wr2, br2):
    wr2c, wr2a = wr2[:C], wr2[C:]
    return pl.pallas_call(
        _mid_body,
        grid=(N // TB,),
        in_specs=[
            pl.BlockSpec((TB, C), lambda i: (i, 0)),
            pl.BlockSpec((TB, C), lambda i: (i, 0)),
            pl.BlockSpec((TB, 1), lambda i: (i, 0)),
            pl.BlockSpec((TB, 1), lambda i: (i, 0)),
            pl.BlockSpec((TB, AH), lambda i: (i, 0)),
            pl.BlockSpec((1, C), lambda i: (0, 0)),
            pl.BlockSpec((1, C), lambda i: (0, 0)),
            pl.BlockSpec((C, E), lambda i: (0, 0)),
            pl.BlockSpec((AH, E), lambda i: (0, 0)),
            pl.BlockSpec((1, E), lambda i: (0, 0)),
            pl.BlockSpec((RCH, RCH), lambda i: (0, 0)),
        ],
        out_specs=[
            pl.BlockSpec((TB, C), lambda i: (i, 0)),
            pl.BlockSpec((TB, C), lambda i: (i, 0)),
            pl.BlockSpec((N, 1), lambda i: (0, 0)),
            pl.BlockSpec((N, 1), lambda i: (0, 0)),
            pl.BlockSpec((1, 1), lambda i: (0, 0)),
            pl.BlockSpec((1, 1), lambda i: (0, 0)),
        ],
        out_shape=[
            jax.ShapeDtypeStruct((N, C), jnp.float32),
            jax.ShapeDtypeStruct((N, C), jnp.float32),
            jax.ShapeDtypeStruct((N, 1), jnp.int32),
            jax.ShapeDtypeStruct((N, 1), jnp.float32),
            jax.ShapeDtypeStruct((1, 1), jnp.float32),
            jax.ShapeDtypeStruct((1, 1), jnp.float32),
        ],
        scratch_shapes=[pltpu.VMEM((N, E), jnp.float32)],
    )(xf, g1, dst1, scale1, a2, n2_g.reshape(1, C), n2_b.reshape(1, C),
      wr2c, wr2a, br2.reshape(1, E), _tril())


def _final_body(x1_ref, g2_ref, dst_ref, scale_ref, out_ref):
    kept = dst_ref[...] < NSLOT
    out_ref[...] = x1_ref[...] + jnp.where(kept, g2_ref[...] * scale_ref[...], 0.0)


def _final(x1, g2, dst2, scale2):
    return pl.pallas_call(
        _final_body,
        grid=(N // TB,),
        in_specs=[
            pl.BlockSpec((TB, C), lambda i: (i, 0)),
            pl.BlockSpec((TB, C), lambda i: (i, 0)),
            pl.BlockSpec((TB, 1), lambda i: (i, 0)),
            pl.BlockSpec((TB, 1), lambda i: (i, 0)),
        ],
        out_specs=pl.BlockSpec((TB, C), lambda i: (i, 0)),
        out_shape=jax.ShapeDtypeStruct((N, C), jnp.float32),
    )(x1, g2, dst2, scale2)


def kernel(x, attr, n1_g, n1_b, w_qkv, b_qkv, wa1, ba1, wr1, br1, wp, bp,
           n2_g, n2_b, wa2, ba2, wr2, br2, w1, b1, w2, b2):
    xf = x.reshape(N, C)

    o, a2_5d, dst1, scale1, z1, bl1 = _window_attention(
        x, attr, n1_g, n1_b, w_qkv, b_qkv, wa1, ba1, wa2, ba2, wr1, br1)
    ot = o.reshape(N, C)
    a2 = a2_5d.reshape(N, AH)

    buf1 = _sc_scatter(ot, dst1.reshape(N // 128, 128))
    ob1 = _expert_proj(buf1, wp, bp)
    g1 = _sc_gather(ob1, dst1.reshape(N // 128, 128))

    x1, xm, dst2, scale2, z2, bl2 = _mid(xf, g1, dst1, scale1, a2,
                                         n2_g, n2_b, wr2, br2)

    buf2 = _sc_scatter(xm, dst2.reshape(N // 128, 128))
    ob2 = _expert_mlp(buf2, w1, b1, w2, b2)
    g2 = _sc_gather(ob2, dst2.reshape(N // 128, 128))

    out = _final(x1, g2, dst2, scale2).reshape(B, Z, H, W, C)
    zs = jnp.stack([z1[0, 0], z2[0, 0]])
    bls = jnp.stack([bl1[0, 0], bl2[0, 0]])
    return (out, zs, bls)


# speedup vs baseline: 1.0173x; 1.0173x over previous
import functools

import jax
import jax.numpy as jnp
from jax import lax
from jax.experimental import pallas as pl
from jax.experimental.pallas import tpu as pltpu
from jax.experimental.pallas import tpu_sc as plsc

B, Z, H, W, C = 1, 4, 32, 64, 256
WZ, WH, WW = 2, 8, 8
NH = 8
E = 64
ATTR = 96
AH = 128
HID = 1024
N = B * Z * H * W
CAP = int(1.25 * N / E)
HD = C // NH
WSZ = WZ * WH * WW
NSLOT = E * CAP

NWORK = 32
CHUNK = N // NWORK
BUF_ROWS = NSLOT + NWORK
RCH = 256
NRCH = N // RCH
TB = 1024


NWPP = H // WH
RB = WZ * H * WW


def _attn_body(x_ref, af_ref, g_ref, b_ref, wqkv_ref, bqkv_ref,
               wa1_ref, ba1_ref, wa2_ref, ba2_ref, wr1c_ref, wr1a_ref,
               br1_ref, tril_ref, o_ref, a2_ref, dst_ref, scale_ref,
               zl_ref, bl_ref, lgacc_ref):
    ones_c8 = jnp.full((C, 8), 1.0 / C, jnp.float32)
    ones_8c = jnp.full((8, C), 1.0 / 8.0, jnp.float32)
    ib = lax.broadcasted_iota(jnp.int32, (NH * WSZ, NH), 0) // WSZ
    jb = lax.broadcasted_iota(jnp.int32, (NH * WSZ, NH), 1)
    onesbd = (ib == jb).astype(jnp.float32)
    ih = lax.broadcasted_iota(jnp.int32, (NH, C), 0)
    jh = lax.broadcasted_iota(jnp.int32, (NH, C), 1) // HD
    rhot = (ih == jh).astype(jnp.float32)

    xb = x_ref[...].reshape(RB, C)
    m8 = jnp.dot(xb, ones_c8, preferred_element_type=jnp.float32)
    e8 = jnp.dot(xb * xb, ones_c8, preferred_element_type=jnp.float32)
    r8 = lax.rsqrt(e8 - m8 * m8 + 1e-5)
    mr = jnp.dot(m8, ones_8c, preferred_element_type=jnp.float32)
    rr = jnp.dot(r8, ones_8c, preferred_element_type=jnp.float32)
    xn = (xb - mr) * rr * g_ref[...] + b_ref[...]
    qkv = jnp.dot(xn, wqkv_ref[...], preferred_element_type=jnp.float32)
    qkv = qkv + bqkv_ref[...]
    q = qkv[:, :C] * (HD ** -0.5)
    k = qkv[:, C:2 * C]
    vv = qkv[:, 2 * C:]
    chunks = [None] * (2 * NWPP)
    for w in range(NWPP):
        r0 = slice(64 * w, 64 * w + 64)
        r1 = slice(RB // 2 + 64 * w, RB // 2 + 64 * w + 64)
        qw = jnp.concatenate([q[r0], q[r1]], axis=0)
        kw = jnp.concatenate([k[r0], k[r1]], axis=0)
        vw = jnp.concatenate([vv[r0], vv[r1]], axis=0)
        exps = []
        outs = []
        for h in range(NH):
            sl = slice(h * HD, (h + 1) * HD)
            qh, kh, vh = qw[:, sl], kw[:, sl], vw[:, sl]
            s = lax.dot_general(qh, kh, (((1,), (1,)), ((), ())),
                                preferred_element_type=jnp.float32)
            eh = jnp.exp(s)
            exps.append(eh)
            outs.append(jnp.dot(eh, vh, preferred_element_type=jnp.float32))
        ecat = jnp.concatenate(exps, axis=1)
        denom = jnp.dot(ecat, onesbd, preferred_element_type=jnp.float32)
        rep = jnp.dot(1.0 / denom, rhot, preferred_element_type=jnp.float32)
        ow = jnp.concatenate(outs, axis=1) * rep
        chunks[w] = ow[:64]
        chunks[NWPP + w] = ow[64:]
    o = jnp.concatenate(chunks, axis=0)
    o_ref[...] = o.reshape(x_ref.shape)
    af = af_ref[...].reshape(RB, ATTR)
    a1 = jax.nn.gelu(jnp.dot(af, wa1_ref[...], preferred_element_type=jnp.float32)
                     + ba1_ref[...])
    a2 = jax.nn.gelu(jnp.dot(af, wa2_ref[...], preferred_element_type=jnp.float32)
                     + ba2_ref[...])
    a2_ref[...] = a2.reshape(a2_ref.shape)
    lg1 = (jnp.dot(o, wr1c_ref[...], preferred_element_type=jnp.float32)
           + jnp.dot(a1, wr1a_ref[...], preferred_element_type=jnp.float32)
           + br1_ref[...])
    i = pl.program_id(0)
    j = pl.program_id(1)
    lgacc_ref[pl.ds(WZ * i, WZ), :, pl.ds(WW * j, WW), :] = (
        lg1.reshape(WZ, H, WW, E))

    @pl.when(jnp.logical_and(i == Z // WZ - 1, j == W // WW - 1))
    def _():
        lg = lgacc_ref[...].reshape(N, E)
        _route_math(lg, tril_ref[...], dst_ref, scale_ref, zl_ref, bl_ref)


def _window_attention(x, attr, n1_g, n1_b, w_qkv, b_qkv,
                      wa1, ba1, wa2, ba2, wr1, br1):
    wr1c, wr1a = wr1[:C], wr1[C:]
    grid = (Z // WZ, W // WW)
    blk = (1, WZ, H, WW, C)
    return pl.pallas_call(
        _attn_body,
        grid=grid,
        in_specs=[
            pl.BlockSpec(blk, lambda i, j: (0, i, 0, j, 0)),
            pl.BlockSpec((1, WZ, H, WW, ATTR), lambda i, j: (0, i, 0, j, 0)),
            pl.BlockSpec((1, C), lambda i, j: (0, 0)),
            pl.BlockSpec((1, C), lambda i, j: (0, 0)),
            pl.BlockSpec((C, 3 * C), lambda i, j: (0, 0)),
            pl.BlockSpec((1, 3 * C), lambda i, j: (0, 0)),
            pl.BlockSpec((ATTR, AH), lambda i, j: (0, 0)),
            pl.BlockSpec((1, AH), lambda i, j: (0, 0)),
            pl.BlockSpec((ATTR, AH), lambda i, j: (0, 0)),
            pl.BlockSpec((1, AH), lambda i, j: (0, 0)),
            pl.BlockSpec((C, E), lambda i, j: (0, 0)),
            pl.BlockSpec((AH, E), lambda i, j: (0, 0)),
            pl.BlockSpec((1, E), lambda i, j: (0, 0)),
            pl.BlockSpec((RCH, RCH), lambda i, j: (0, 0)),
        ],
        out_specs=[
            pl.BlockSpec(blk, lambda i, j: (0, i, 0, j, 0)),
            pl.BlockSpec((1, WZ, H, WW, AH), lambda i, j: (0, i, 0, j, 0)),
            pl.BlockSpec((N, 1), lambda i, j: (0, 0)),
            pl.BlockSpec((N, 1), lambda i, j: (0, 0)),
            pl.BlockSpec((1, 1), lambda i, j: (0, 0)),
            pl.BlockSpec((1, 1), lambda i, j: (0, 0)),
        ],
        out_shape=[
            jax.ShapeDtypeStruct((B, Z, H, W, C), jnp.float32),
            jax.ShapeDtypeStruct((B, Z, H, W, AH), jnp.float32),
            jax.ShapeDtypeStruct((N, 1), jnp.int32),
            jax.ShapeDtypeStruct((N, 1), jnp.float32),
            jax.ShapeDtypeStruct((1, 1), jnp.float32),
            jax.ShapeDtypeStruct((1, 1), jnp.float32),
        ],
        scratch_shapes=[pltpu.VMEM((Z, H, W, E), jnp.float32)],
    )(x, attr, n1_g.reshape(1, C), n1_b.reshape(1, C), w_qkv,
      b_qkv.reshape(1, 3 * C), wa1, ba1.reshape(1, AH), wa2,
      ba2.reshape(1, AH), wr1c, wr1a, br1.reshape(1, E), _tril())


def _route_math(lg, tril, dst_ref, scale_ref, zl_ref, bl_ref):
    m = jnp.max(lg, -1, keepdims=True)
    ex = jnp.exp(lg - m)
    se = jnp.sum(ex, -1, keepdims=True)
    probs = ex / se
    lse = jnp.log(se) + m
    zl_ref[...] = jnp.sum(lse * lse, 0, keepdims=True) * (1.0 / N)
    p = jnp.max(probs, -1, keepdims=True)
    lanes = lax.broadcasted_iota(jnp.int32, (N, E), 1)
    eidx = jnp.min(jnp.where(probs >= p, lanes, E), -1, keepdims=True)
    oh = (lanes == eidx).astype(jnp.float32)
    bl_ref[...] = (float(E) / (float(N) * float(N))) * jnp.sum(
        jnp.sum(oh, 0, keepdims=True) * jnp.sum(probs, 0, keepdims=True),
        1, keepdims=True)
    scale_ref[...] = p
    off = jnp.zeros((1, E), jnp.float32)
    for cidx in range(NRCH):
        rows = slice(cidx * RCH, (cidx + 1) * RCH)
        ohc = oh[rows]
        cum = jnp.dot(tril, ohc, preferred_element_type=jnp.float32) + off
        pos = jnp.sum(cum * ohc, -1, keepdims=True) - 1.0
        keep = pos < float(CAP)
        posi = pos.astype(jnp.int32)
        ec = eidx[rows]
        dst = jnp.where(keep, ec * CAP + posi, NSLOT + cidx)
        dst_ref[rows, :] = dst
        off = cum[RCH - 1:RCH, :]


_TRIL = None


def _tril():
    global _TRIL
    if _TRIL is None:
        _TRIL = jnp.tril(jnp.ones((RCH, RCH), jnp.float32))
    return _TRIL


_SC_MESH = None


def _sc_mesh():
    global _SC_MESH
    if _SC_MESH is None:
        _SC_MESH = plsc.VectorSubcoreMesh(core_axis_name="c", subcore_axis_name="s")
    return _SC_MESH


@functools.partial(jax.jit)
def _sc_scatter(tok, dst2d):
    @functools.partial(
        pl.kernel,
        out_type=jax.ShapeDtypeStruct((BUF_ROWS, C), jnp.float32),
        mesh=_sc_mesh(),
        scratch_types=[
            pltpu.VMEM((2, 128), jnp.int32),
            pltpu.VMEM((128, C), jnp.float32),
            pltpu.SemaphoreType.DMA,
        ],
    )
    def scatter_k(tok_hbm, dst_hbm, buf_hbm, idx_v, rows_v, sem):
        wid = lax.axis_index("s") * 2 + lax.axis_index("c")
        base = wid * CHUNK
        pltpu.sync_copy(dst_hbm.at[pl.ds(wid * 2, 2)], idx_v)
        for j in range(2):
            pltpu.sync_copy(tok_hbm.at[pl.ds(base + j * 128, 128)], rows_v)
            pltpu.async_copy(rows_v, buf_hbm.at[idx_v.at[j]], sem).wait()

    return scatter_k(tok, dst2d)


@functools.partial(jax.jit)
def _sc_gather(buf, dst2d):
    @functools.partial(
        pl.kernel,
        out_type=jax.ShapeDtypeStruct((N, C), jnp.float32),
        mesh=_sc_mesh(),
        scratch_types=[
            pltpu.VMEM((2, 128), jnp.int32),
            pltpu.VMEM((128, C), jnp.float32),
            pltpu.SemaphoreType.DMA,
        ],
    )
    def gather_k(buf_hbm, dst_hbm, out_hbm, idx_v, rows_v, sem):
        wid = lax.axis_index("s") * 2 + lax.axis_index("c")
        base = wid * CHUNK
        pltpu.sync_copy(dst_hbm.at[pl.ds(wid * 2, 2)], idx_v)
        for j in range(2):
            pltpu.async_copy(buf_hbm.at[idx_v.at[j]], rows_v, sem).wait()
            pltpu.sync_copy(rows_v, out_hbm.at[pl.ds(base + j * 128, 128)])

    return gather_k(buf, dst2d)


def _exp1_body(buf_ref, wp_ref, bp_ref, ob_ref):
    e = pl.program_id(0)
    ob_ref[...] = (jnp.dot(buf_ref[...], wp_ref[0],
                           preferred_element_type=jnp.float32)
                   + bp_ref[pl.ds(e, 1), :])


def _expert_proj(buf, wp, bp):
    return pl.pallas_call(
        _exp1_body,
        grid=(E,),
        in_specs=[
            pl.BlockSpec((CAP, C), lambda e: (e, 0)),
            pl.BlockSpec((1, C, C), lambda e: (e, 0, 0)),
            pl.BlockSpec((E, C), lambda e: (0, 0)),
        ],
        out_specs=pl.BlockSpec((CAP, C), lambda e: (e, 0)),
        out_shape=jax.ShapeDtypeStruct((BUF_ROWS, C), jnp.float32),
    )(buf, wp, bp)


def _exp2_body(buf_ref, w1_ref, b1_ref, w2_ref, b2_ref, ob_ref):
    e = pl.program_id(0)
    h = jax.nn.gelu(jnp.dot(buf_ref[...], w1_ref[0],
                            preferred_element_type=jnp.float32)
                    + b1_ref[pl.ds(e, 1), :])
    ob_ref[...] = (jnp.dot(h, w2_ref[0], preferred_element_type=jnp.float32)
                   + b2_ref[pl.ds(e, 1), :])


def _expert_mlp(buf, w1, b1, w2, b2):
    return pl.pallas_call(
        _exp2_body,
        grid=(E,),
        in_specs=[
            pl.BlockSpec((CAP, C), lambda e: (e, 0)),
            pl.BlockSpec((1, C, HID), lambda e: (e, 0, 0)),
            pl.BlockSpec((E, HID), lambda e: (0, 0)),
            pl.BlockSpec((1, HID, C), lambda e: (e, 0, 0)),
            pl.BlockSpec((E, C), lambda e: (0, 0)),
        ],
        out_specs=pl.BlockSpec((CAP, C), lambda e: (e, 0)),
        out_shape=jax.ShapeDtypeStruct((BUF_ROWS, C), jnp.float32),
    )(buf, w1, b1, w2, b2)


def _mid_body(x_ref, g1_ref, dst_ref, scale_ref, a2_ref, n2g_ref, n2b_ref,
              wr2c_ref, wr2a_ref, br2_ref, tril_ref, x1_ref, xm_ref,
              dst2_ref, scale2_ref, zl_ref, bl_ref, lgacc_ref):
    kept = dst_ref[...] < NSLOT
    y1 = jnp.where(kept, g1_ref[...] * scale_ref[...], 0.0)
    x1 = x_ref[...] + y1
    x1_ref[...] = x1
    ones_c8 = jnp.full((C, 8), 1.0 / C, jnp.float32)
    ones_8c = jnp.full((8, C), 1.0 / 8.0, jnp.float32)
    m8 = jnp.dot(x1, ones_c8, preferred_element_type=jnp.float32)
    e8 = jnp.dot(x1 * x1, ones_c8, preferred_element_type=jnp.float32)
    r8 = lax.rsqrt(e8 - m8 * m8 + 1e-5)
    mr = jnp.dot(m8, ones_8c, preferred_element_type=jnp.float32)
    rr = jnp.dot(r8, ones_8c, preferred_element_type=jnp.float32)
    xm = (x1 - mr) * rr * n2g_ref[...] + n2b_ref[...]
    xm_ref[...] = xm
    lg2 = (jnp.dot(xm, wr2c_ref[...], preferred_element_type=jnp.float32)
           + jnp.dot(a2_ref[...], wr2a_ref[...], preferred_element_type=jnp.float32)
           + br2_ref[...])
    i = pl.program_id(0)
    lgacc_ref[pl.ds(i * TB, TB), :] = lg2

    @pl.when(i == N // TB - 1)
    def _():
        _route_math(lgacc_ref[...], tril_ref[...], dst2_ref, scale2_ref,
                    zl_ref, bl_ref)


def _mid(xf, g1, dst1, scale1, a2, n2_g, n2_b, wr2, br2):
    wr2c, wr2a = wr2[:C], wr2[C:]
    return pl.pallas_call(
        _mid_body,
        grid=(N // TB,),
        in_specs=[
            pl.BlockSpec((TB, C), lambda i: (i, 0)),
            pl.BlockSpec((TB, C), lambda i: (i, 0)),
            pl.BlockSpec((TB, 1), lambda i: (i, 0)),
            pl.BlockSpec((TB, 1), lambda i: (i, 0)),
            pl.BlockSpec((TB, AH), lambda i: (i, 0)),
            pl.BlockSpec((1, C), lambda i: (0, 0)),
            pl.BlockSpec((1, C), lambda i: (0, 0)),
            pl.BlockSpec((C, E), lambda i: (0, 0)),
            pl.BlockSpec((AH, E), lambda i: (0, 0)),
            pl.BlockSpec((1, E), lambda i: (0, 0)),
            pl.BlockSpec((RCH, RCH), lambda i: (0, 0)),
        ],
        out_specs=[
            pl.BlockSpec((TB, C), lambda i: (i, 0)),
            pl.BlockSpec((TB, C), lambda i: (i, 0)),
            pl.BlockSpec((N, 1), lambda i: (0, 0)),
            pl.BlockSpec((N, 1), lambda i: (0, 0)),
            pl.BlockSpec((1, 1), lambda i: (0, 0)),
            pl.BlockSpec((1, 1), lambda i: (0, 0)),
        ],
        out_shape=[
            jax.ShapeDtypeStruct((N, C), jnp.float32),
            jax.ShapeDtypeStruct((N, C), jnp.float32),
            jax.ShapeDtypeStruct((N, 1), jnp.int32),
            jax.ShapeDtypeStruct((N, 1), jnp.float32),
            jax.ShapeDtypeStruct((1, 1), jnp.float32),
            jax.ShapeDtypeStruct((1, 1), jnp.float32),
        ],
        scratch_shapes=[pltpu.VMEM((N, E), jnp.float32)],
    )(xf, g1, dst1, scale1, a2, n2_g.reshape(1, C), n2_b.reshape(1, C),
      wr2c, wr2a, br2.reshape(1, E), _tril())


def _final_body(x1_ref, g2_ref, dst_ref, scale_ref, out_ref):
    kept = dst_ref[...] < NSLOT
    out_ref[...] = x1_ref[...] + jnp.where(kept, g2_ref[...] * scale_ref[...], 0.0)


def _final(x1, g2, dst2, scale2):
    return pl.pallas_call(
        _final_body,
        grid=(N // TB,),
        in_specs=[
            pl.BlockSpec((TB, C), lambda i: (i, 0)),
            pl.BlockSpec((TB, C), lambda i: (i, 0)),
            pl.BlockSpec((TB, 1), lambda i: (i, 0)),
            pl.BlockSpec((TB, 1), lambda i: (i, 0)),
        ],
        out_specs=pl.BlockSpec((TB, C), lambda i: (i, 0)),
        out_shape=jax.ShapeDtypeStruct((N, C), jnp.float32),
    )(x1, g2, dst2, scale2)


def kernel(x, attr, n1_g, n1_b, w_qkv, b_qkv, wa1, ba1, wr1, br1, wp, bp,
           n2_g, n2_b, wa2, ba2, wr2, br2, w1, b1, w2, b2):
    xf = x.reshape(N, C)

    o, a2_5d, dst1, scale1, z1, bl1 = _window_attention(
        x, attr, n1_g, n1_b, w_qkv, b_qkv, wa1, ba1, wa2, ba2, wr1, br1)
    ot = o.reshape(N, C)
    a2 = a2_5d.reshape(N, AH)

    buf1 = _sc_scatter(ot, dst1.reshape(N // 128, 128))
    ob1 = _expert_proj(buf1, wp, bp)
    g1 = _sc_gather(ob1, dst1.reshape(N // 128, 128))

    x1, xm, dst2, scale2, z2, bl2 = _mid(xf, g1, dst1, scale1, a2,
                                         n2_g, n2_b, wr2, br2)

    buf2 = _sc_scatter(xm, dst2.reshape(N // 128, 128))
    ob2 = _expert_mlp(buf2, w1, b1, w2, b2)
    g2 = _sc_gather(ob2, dst2.reshape(N // 128, 128))

    out = _final(x1, g2, dst2, scale2).reshape(B, Z, H, W, C)
    zs = jnp.stack([z1[0, 0], z2[0, 0]])
    bls = jnp.stack([bl1[0, 0], bl2[0, 0]])
    return (out, zs, bls)
